# baseline (speedup 1.0000x reference)
import jax
import jax.numpy as jnp
from jax import lax
from jax.experimental import pallas as pl
from jax.experimental.pallas import tpu as pltpu
from jax.experimental.pallas import tpu_sc as plsc

_BATCH = 16384
_FACTORS = 64
_LANES = 16
_NUM_CORES = 2
_NUM_SUBCORES = 16
_NW = _NUM_CORES * _NUM_SUBCORES
_CHUNK = _BATCH // _NW
_W = 32
_NWIN = _CHUNK // _W
_SUB = 8


def _mf_body(user_ix_hbm, item_ix_hbm, user_emb_hbm, item_emb_hbm, out_hbm,
             uix_v, iix_v, ug, vg, out2d, sem_u, sem_v):
    wid = lax.axis_index("s") * _NUM_CORES + lax.axis_index("c")
    base = wid * _CHUNK
    pltpu.sync_copy(user_ix_hbm.at[pl.ds(base, _CHUNK)], uix_v)
    pltpu.sync_copy(item_ix_hbm.at[pl.ds(base, _CHUNK)], iix_v)
    for w in range(_NWIN):
        def fire(b, carry):
            uvec = lax.shift_right_logical(uix_v[pl.ds(w * _W + b * _LANES, _LANES)], 3)
            ivec = lax.shift_right_logical(iix_v[pl.ds(w * _W + b * _LANES, _LANES)], 3)
            for t in range(_LANES):
                j = b * _LANES + t
                pltpu.async_copy(user_emb_hbm.at[pl.ds(uvec[t] * 8, 8)],
                                 ug.at[j], sem_u)
                pltpu.async_copy(item_emb_hbm.at[pl.ds(ivec[t] * 8, 8)],
                                 vg.at[j], sem_v)
            return carry

        lax.fori_loop(0, _W // _LANES, fire, 0)

        def drain(j, carry):
            pltpu.make_async_copy(user_emb_hbm.at[pl.ds(0, 8)],
                                  ug.at[j], sem_u).wait()
            pltpu.make_async_copy(item_emb_hbm.at[pl.ds(0, 8)],
                                  vg.at[j], sem_v).wait()
            return carry

        lax.fori_loop(0, _W, drain, 0)

        def mul(b, carry):
            su = lax.rem(uix_v[pl.ds(w * _W + b * _LANES, _LANES)], 8)
            sv = lax.rem(iix_v[pl.ds(w * _W + b * _LANES, _LANES)], 8)
            for t in range(_LANES):
                j = b * _LANES + t
                for k in range(_FACTORS // _LANES):
                    sl = pl.ds(k * _LANES, _LANES)
                    out2d[j, sl] = ug[j, su[t], sl] * vg[j, sv[t], sl]
            return carry

        lax.fori_loop(0, _W // _LANES, mul, 0)
        pltpu.sync_copy(out2d, out_hbm.at[pl.ds(base + w * _W, _W)])


def kernel(user_ix, item_ix, user_emb, item_emb):
    uix = user_ix.reshape(-1)
    iix = item_ix.reshape(-1)
    mesh = plsc.VectorSubcoreMesh(core_axis_name="c", subcore_axis_name="s")
    run = pl.kernel(
        _mf_body,
        mesh=mesh,
        compiler_params=pltpu.CompilerParams(use_tc_tiling_on_sc=True),
        out_type=jax.ShapeDtypeStruct((_BATCH, _FACTORS), jnp.float32),
        scratch_types=[
            pltpu.VMEM((_CHUNK,), jnp.int32),
            pltpu.VMEM((_CHUNK,), jnp.int32),
            pltpu.VMEM((_W, _SUB, _FACTORS), jnp.float32),
            pltpu.VMEM((_W, _SUB, _FACTORS), jnp.float32),
            pltpu.VMEM((_W, _FACTORS), jnp.float32),
            pltpu.SemaphoreType.DMA,
            pltpu.SemaphoreType.DMA,
        ],
    )
    return run(uix, iix, user_emb, item_emb)


# speedup vs baseline: 1.0333x; 1.0333x over previous
import jax
import jax.numpy as jnp
from jax import lax
from jax.experimental import pallas as pl
from jax.experimental.pallas import tpu as pltpu
from jax.experimental.pallas import tpu_sc as plsc

_BATCH = 16384
_FACTORS = 64
_LANES = 16
_NUM_CORES = 2
_NUM_SUBCORES = 16
_NW = _NUM_CORES * _NUM_SUBCORES
_CHUNK = _BATCH // _NW
_W = 16
_NWIN = _CHUNK // _W
_SUB = 8


def _mf_body(user_ix_hbm, item_ix_hbm, user_emb_hbm, item_emb_hbm, out_hbm,
             uix_v, iix_v, ug0, vg0, ug1, vg1, out2d, sem_u, sem_v):
    wid = lax.axis_index("s") * _NUM_CORES + lax.axis_index("c")
    base = wid * _CHUNK
    pltpu.sync_copy(user_ix_hbm.at[pl.ds(base, _CHUNK)], uix_v)
    pltpu.sync_copy(item_ix_hbm.at[pl.ds(base, _CHUNK)], iix_v)

    def fire(w, ug, vg):
        uvec = lax.shift_right_logical(uix_v[pl.ds(w * _W, _LANES)], 3)
        ivec = lax.shift_right_logical(iix_v[pl.ds(w * _W, _LANES)], 3)
        for t in range(_LANES):
            pltpu.async_copy(user_emb_hbm.at[pl.ds(uvec[t] * 8, 8)],
                             ug.at[t], sem_u)
            pltpu.async_copy(item_emb_hbm.at[pl.ds(ivec[t] * 8, 8)],
                             vg.at[t], sem_v)

    def drain_mul_out(w, ug, vg):
        def drain(j, carry):
            pltpu.make_async_copy(user_emb_hbm.at[pl.ds(0, 8)],
                                  ug.at[j], sem_u).wait()
            pltpu.make_async_copy(item_emb_hbm.at[pl.ds(0, 8)],
                                  vg.at[j], sem_v).wait()
            return carry

        lax.fori_loop(0, _W, drain, 0)
        su = lax.rem(uix_v[pl.ds(w * _W, _LANES)], 8)
        sv = lax.rem(iix_v[pl.ds(w * _W, _LANES)], 8)
        for t in range(_LANES):
            for k in range(_FACTORS // _LANES):
                sl = pl.ds(k * _LANES, _LANES)
                out2d[t, sl] = ug[t, su[t], sl] * vg[t, sv[t], sl]
        pltpu.sync_copy(out2d, out_hbm.at[pl.ds(base + w * _W, _W)])

    fire(0, ug0, vg0)

    def pair(p, carry):
        w0 = p * 2
        fire(w0 + 1, ug1, vg1)
        drain_mul_out(w0, ug0, vg0)

        @pl.when(p + 1 < _NWIN // 2)
        def _():
            fire(w0 + 2, ug0, vg0)

        drain_mul_out(w0 + 1, ug1, vg1)
        return carry

    lax.fori_loop(0, _NWIN // 2, pair, 0)


def kernel(user_ix, item_ix, user_emb, item_emb):
    uix = user_ix.reshape(-1)
    iix = item_ix.reshape(-1)
    mesh = plsc.VectorSubcoreMesh(core_axis_name="c", subcore_axis_name="s")
    run = pl.kernel(
        _mf_body,
        mesh=mesh,
        compiler_params=pltpu.CompilerParams(use_tc_tiling_on_sc=True),
        out_type=jax.ShapeDtypeStruct((_BATCH, _FACTORS), jnp.float32),
        scratch_types=[
            pltpu.VMEM((_CHUNK,), jnp.int32),
            pltpu.VMEM((_CHUNK,), jnp.int32),
            pltpu.VMEM((_W, _SUB, _FACTORS), jnp.float32),
            pltpu.VMEM((_W, _SUB, _FACTORS), jnp.float32),
            pltpu.VMEM((_W, _SUB, _FACTORS), jnp.float32),
            pltpu.VMEM((_W, _SUB, _FACTORS), jnp.float32),
            pltpu.VMEM((_W, _FACTORS), jnp.float32),
            pltpu.SemaphoreType.DMA,
            pltpu.SemaphoreType.DMA,
        ],
    )
    return run(uix, iix, user_emb, item_emb)
